# Initial kernel scaffold; baseline (speedup 1.0000x reference)
import jax
import jax.numpy as jnp
from jax.experimental import pallas as pl


def kernel(x, table, W1, b1, W2, b2):
    raise NotImplementedError("write your pallas kernel here")


# speedup vs baseline: 17.6215x; 17.6215x over previous
import functools

import jax
import jax.numpy as jnp
from jax import lax
from jax.experimental import pallas as pl
from jax.experimental.pallas import tpu as pltpu
from jax.experimental.pallas import tpu_sc as plsc

_B, _S, _V, _E = 4096, 200, 100000, 64
_NC, _NS = 2, 16
_NW = _NC * _NS
_BPW = _B // _NW
_C0, _C1 = 128, 72
_NSLOT = 4


def _issue_row(table_hbm, idx_v, i, slot_buf, sem):
    off = i * _S
    pltpu.async_copy(table_hbm.at[idx_v.at[pl.ds(off, _C0)]],
                     slot_buf.at[pl.ds(0, _C0)], sem)
    pltpu.async_copy(table_hbm.at[idx_v.at[pl.ds(off + _C0, _C1)]],
                     slot_buf.at[pl.ds(_C0, _C1)], sem)


def _drain_row(table_hbm, slot_buf, sem):
    pltpu.make_async_copy(table_hbm.at[pl.ds(0, _S)], slot_buf, sem).wait()


def _reduce_row(buf, i, avg_v):
    def body(j, acc):
        accs = list(acc)
        for u in range(4):
            r = j * 4 + u
            for g in range(_E // 16):
                accs[g] = accs[g] + buf[r, pl.ds(g * 16, 16)]
        return tuple(accs)

    z = jnp.zeros((16,), jnp.float32)
    acc = lax.fori_loop(0, _S // 4, body, (z,) * (_E // 16))
    for g in range(_E // 16):
        avg_v[i, pl.ds(g * 16, 16)] = acc[g] * (1.0 / _S)


def _sc_body(x_hbm, table_hbm, out_hbm, idx_v, bufs, avg_v, sems):
    wid = lax.axis_index("s") * _NC + lax.axis_index("c")
    base = wid * _BPW
    pltpu.sync_copy(x_hbm.at[pl.ds(base * _S, _BPW * _S)], idx_v)

    for j in range(_NSLOT - 1):
        _issue_row(table_hbm, idx_v, j, bufs[j], sems[j])

    def outer(k, _):
        for u in range(_NSLOT):
            i = k * _NSLOT + u
            _drain_row(table_hbm, bufs[u], sems[u])
            nxt = i + _NSLOT - 1
            nxt_slot = (u + _NSLOT - 1) % _NSLOT

            @pl.when(nxt < _BPW)
            def _():
                _issue_row(table_hbm, idx_v, nxt, bufs[nxt_slot], sems[nxt_slot])

            _reduce_row(bufs[u], i, avg_v)
        return 0

    lax.fori_loop(0, _BPW // _NSLOT, outer, 0)
    pltpu.sync_copy(avg_v, out_hbm.at[pl.ds(base, _BPW)])


def _sc_gather_mean(x_flat, table):
    mesh = plsc.VectorSubcoreMesh(core_axis_name="c", subcore_axis_name="s",
                                  num_cores=_NC, num_subcores=_NS)
    fn = pl.kernel(
        _sc_body,
        out_type=jax.ShapeDtypeStruct((_B, _E), jnp.float32),
        mesh=mesh,
        scratch_types=[
            pltpu.VMEM((_BPW * _S,), jnp.int32),
            [pltpu.VMEM((_S, _E), jnp.float32) for _ in range(_NSLOT)],
            pltpu.VMEM((_BPW, _E), jnp.float32),
            [pltpu.SemaphoreType.DMA for _ in range(_NSLOT)],
        ],
        compiler_params=pltpu.CompilerParams(use_tc_tiling_on_sc=False),
    )
    return fn(x_flat, table)


def _mlp_body(avg_ref, w1_ref, b1_ref, w2_ref, b2_ref, out_ref):
    h = jnp.dot(avg_ref[...], w1_ref[...],
                preferred_element_type=jnp.float32) + b1_ref[...]
    h = jnp.maximum(h, 0.0)
    logits = jnp.dot(h, w2_ref[...],
                     preferred_element_type=jnp.float32) + b2_ref[...]
    mx = jnp.max(logits, axis=1, keepdims=True)
    lse = jnp.log(jnp.sum(jnp.exp(logits - mx), axis=1, keepdims=True)) + mx
    out_ref[...] = logits - lse


def _tc_mlp(avg, W1, b1, W2, b2):
    return pl.pallas_call(
        _mlp_body,
        out_shape=jax.ShapeDtypeStruct((_B, 2), jnp.float32),
    )(avg, W1, b1.reshape(1, -1), W2, b2.reshape(1, -1))


def kernel(x, table, W1, b1, W2, b2):
    x_flat = x.reshape(-1).astype(jnp.int32)
    avg = _sc_gather_mean(x_flat, table)
    return _tc_mlp(avg, W1, b1, W2, b2)
